# Initial kernel scaffold; baseline (speedup 1.0000x reference)
import jax
import jax.numpy as jnp
from jax.experimental import pallas as pl


def kernel(x, edge_index, batch, c0_W1, c0_b1, c0_g1, c0_be1, c0_W2, c0_b2, bn0_g, bn0_b, c1_W1, c1_b1, c1_g1, c1_be1, c1_W2, c1_b2, bn1_g, bn1_b, l0_W, l0_b, l1_W, l1_b, l2_W, l2_b):
    raise NotImplementedError("write your pallas kernel here")


# speedup vs baseline: 2.6980x; 2.6980x over previous
import functools

import jax
import jax.numpy as jnp
from jax import lax
from jax.experimental import pallas as pl
from jax.experimental.pallas import tpu as pltpu
from jax.experimental.pallas import tpu_sc as plsc

N_NODES = 10000
N_PAD = 10240
N_EDGES = 320000
N_GRAPHS = 64
BN_EPS = 1e-5

NT = 16
K_EDGE = 128
HALF = N_PAD // 2
RPT = HALF // NT
ROW_CH = 64
N_RCH = RPT // ROW_CH
GARB = 8


def _sc_phases(table_init, table_gather, out, dst3_passes, t,
               src_v, dst_v, rows_v, stage_v, acc, sem, s, nch):
    row0 = s * RPT
    for p in range(2):
        base = p * HALF
        pltpu.sync_copy(dst3_passes[p].at[t], dst_v)

        def stage(k, carry):
            r = row0 + k * ROW_CH
            pltpu.sync_copy(table_init.at[pl.ds(base + r, ROW_CH)], stage_v)
            pltpu.sync_copy(stage_v, acc.at[pl.ds(r, ROW_CH)])
            return carry
        lax.fori_loop(0, N_RCH, stage, 0)
        plsc.subcore_barrier()

        def chunk(j, carry):
            pltpu.async_copy(table_gather.at[src_v.at[j]], rows_v, sem).wait()
            pltpu.sync_copy(rows_v, acc.at[dst_v.at[j]], add=True)
            return carry
        lax.fori_loop(0, nch, chunk, 0)
        plsc.subcore_barrier()

        def wout(k, carry):
            r = row0 + k * ROW_CH
            pltpu.sync_copy(acc.at[pl.ds(r, ROW_CH)], stage_v)
            pltpu.sync_copy(stage_v, out.at[pl.ds(base + r, ROW_CH)])
            return carry
        lax.fori_loop(0, N_RCH, wout, 0)


def _make_sc_aggr_edgesplit(n, d, e):
    per_tile = e // (2 * NT)
    nch = -(-per_tile // K_EDGE)

    mesh = plsc.VectorSubcoreMesh(core_axis_name="c", subcore_axis_name="s")

    @functools.partial(
        pl.kernel,
        out_type=(jax.ShapeDtypeStruct((n, d), jnp.float32),
                  jax.ShapeDtypeStruct((n, d), jnp.float32)),
        mesh=mesh,
        scratch_types=[
            pltpu.VMEM((nch, K_EDGE), jnp.int32),
            pltpu.VMEM((nch, K_EDGE), jnp.int32),
            pltpu.VMEM((K_EDGE, d), jnp.float32),
            pltpu.VMEM((ROW_CH, d), jnp.float32),
            pltpu.VMEM_SHARED((HALF + GARB, d), jnp.float32),
            pltpu.SemaphoreType.DMA,
        ],
    )
    def aggr(table, zeros, src3, dst3a, dst3b, out_a, out_b,
             src_v, dst_v, rows_v, stage_v, acc, sem):
        c = lax.axis_index("c")
        s = lax.axis_index("s")
        wid = c * NT + s
        pltpu.sync_copy(src3.at[wid], src_v)

        @pl.when(c == 0)
        def _():
            _sc_phases(table, table, out_a, (dst3a, dst3b), wid,
                       src_v, dst_v, rows_v, stage_v, acc, sem, s, nch)

        @pl.when(c == 1)
        def _():
            _sc_phases(zeros, table, out_b, (dst3a, dst3b), wid,
                       src_v, dst_v, rows_v, stage_v, acc, sem, s, nch)

    return aggr


def _make_sc_aggr_featsplit(n, d, e):
    per_tile = e // NT
    nch = -(-per_tile // K_EDGE)

    mesh = plsc.VectorSubcoreMesh(core_axis_name="c", subcore_axis_name="s")

    @functools.partial(
        pl.kernel,
        out_type=(jax.ShapeDtypeStruct((n, d), jnp.float32),
                  jax.ShapeDtypeStruct((n, d), jnp.float32)),
        mesh=mesh,
        scratch_types=[
            pltpu.VMEM((nch, K_EDGE), jnp.int32),
            pltpu.VMEM((nch, K_EDGE), jnp.int32),
            pltpu.VMEM((K_EDGE, d), jnp.float32),
            pltpu.VMEM((ROW_CH, d), jnp.float32),
            pltpu.VMEM_SHARED((HALF + GARB, d), jnp.float32),
            pltpu.SemaphoreType.DMA,
        ],
    )
    def aggr(tlo, thi, src3, dst3a, dst3b, out_lo, out_hi,
             src_v, dst_v, rows_v, stage_v, acc, sem):
        c = lax.axis_index("c")
        s = lax.axis_index("s")
        pltpu.sync_copy(src3.at[s], src_v)

        @pl.when(c == 0)
        def _():
            _sc_phases(tlo, tlo, out_lo, (dst3a, dst3b), s,
                       src_v, dst_v, rows_v, stage_v, acc, sem, s, nch)

        @pl.when(c == 1)
        def _():
            _sc_phases(thi, thi, out_hi, (dst3a, dst3b), s,
                       src_v, dst_v, rows_v, stage_v, acc, sem, s, nch)

    return aggr


_R = 512


def _mlp0_body(za, zb, w1, b1, w2, b2, olo, ohi):
    z = za[...] + zb[...]
    t = jnp.dot(z, w1[...], preferred_element_type=jnp.float32) + b1[...]
    t = jnp.maximum(t, 0.0)
    h = jnp.dot(t, w2[...], preferred_element_type=jnp.float32) + b2[...]
    h = jnp.maximum(h, 0.0)
    olo[...] = h[:, :128]
    ohi[...] = h[:, 128:]


def _tc_mlp0(z_a, z_b, w1, b1, w2, b2):
    n = z_a.shape[0]
    grid = n // _R
    return pl.pallas_call(
        _mlp0_body,
        grid=(grid,),
        in_specs=[
            pl.BlockSpec((_R, 128), lambda i: (i, 0)),
            pl.BlockSpec((_R, 128), lambda i: (i, 0)),
            pl.BlockSpec((128, 256), lambda i: (0, 0)),
            pl.BlockSpec((1, 256), lambda i: (0, 0)),
            pl.BlockSpec((256, 256), lambda i: (0, 0)),
            pl.BlockSpec((1, 256), lambda i: (0, 0)),
        ],
        out_specs=[
            pl.BlockSpec((_R, 128), lambda i: (i, 0)),
            pl.BlockSpec((_R, 128), lambda i: (i, 0)),
        ],
        out_shape=[
            jax.ShapeDtypeStruct((n, 128), jnp.float32),
            jax.ShapeDtypeStruct((n, 128), jnp.float32),
        ],
    )(z_a, z_b, w1, b1, w2, b2)


def _mlp1_pool_body(x, h1lo, h1hi, z1lo, z1hi, bvec, w1lo, w1hi, b1, w2, b2,
                    l0w, l1w, l2w, bsum, out, p0, p1, p2):
    i = pl.program_id(0)
    nb = pl.num_programs(0)

    @pl.when(i == 0)
    def _():
        p0[...] = jnp.zeros_like(p0)
        p1[...] = jnp.zeros_like(p1)
        p2[...] = jnp.zeros_like(p2)

    t = (jnp.dot(z1lo[...], w1lo[...], preferred_element_type=jnp.float32)
         + jnp.dot(z1hi[...], w1hi[...], preferred_element_type=jnp.float32)
         + b1[...])
    t = jnp.maximum(t, 0.0)
    h2 = jnp.dot(t, w2[...], preferred_element_type=jnp.float32) + b2[...]
    h2 = jnp.maximum(h2, 0.0)

    b = bvec[0, 0, :]
    pm = (b[None, :] == lax.broadcasted_iota(jnp.int32, (N_GRAPHS, _R), 0)
          ).astype(jnp.float32)
    h1 = jnp.concatenate([h1lo[...], h1hi[...]], axis=1)
    p0[...] += jnp.dot(pm, x[...], preferred_element_type=jnp.float32)
    p1[...] += jnp.dot(pm, h1, preferred_element_type=jnp.float32)
    p2[...] += jnp.dot(pm, h2, preferred_element_type=jnp.float32)

    @pl.when(i == nb - 1)
    def _():
        out[...] = (
            jnp.dot(p0[...], l0w[...], preferred_element_type=jnp.float32)
            + jnp.dot(p1[...], l1w[...], preferred_element_type=jnp.float32)
            + jnp.dot(p2[...], l2w[...], preferred_element_type=jnp.float32)
            + bsum[...])


def _tc_mlp1_pool(x, h1lo, h1hi, z1lo, z1hi, batch3, w1lo, w1hi, b1, w2, b2,
                  l0w, l1w, l2w, bsum):
    n = x.shape[0]
    grid = n // _R
    return pl.pallas_call(
        _mlp1_pool_body,
        grid=(grid,),
        in_specs=[
            pl.BlockSpec((_R, 128), lambda i: (i, 0)),
            pl.BlockSpec((_R, 128), lambda i: (i, 0)),
            pl.BlockSpec((_R, 128), lambda i: (i, 0)),
            pl.BlockSpec((_R, 128), lambda i: (i, 0)),
            pl.BlockSpec((_R, 128), lambda i: (i, 0)),
            pl.BlockSpec((1, 1, _R), lambda i: (i, 0, 0)),
            pl.BlockSpec((128, 256), lambda i: (0, 0)),
            pl.BlockSpec((128, 256), lambda i: (0, 0)),
            pl.BlockSpec((1, 256), lambda i: (0, 0)),
            pl.BlockSpec((256, 256), lambda i: (0, 0)),
            pl.BlockSpec((1, 256), lambda i: (0, 0)),
            pl.BlockSpec((128, 128), lambda i: (0, 0)),
            pl.BlockSpec((256, 128), lambda i: (0, 0)),
            pl.BlockSpec((256, 128), lambda i: (0, 0)),
            pl.BlockSpec((1, 128), lambda i: (0, 0)),
        ],
        out_specs=pl.BlockSpec((N_GRAPHS, 128), lambda i: (0, 0)),
        out_shape=jax.ShapeDtypeStruct((N_GRAPHS, 128), jnp.float32),
        scratch_shapes=[
            pltpu.VMEM((N_GRAPHS, 128), jnp.float32),
            pltpu.VMEM((N_GRAPHS, 256), jnp.float32),
            pltpu.VMEM((N_GRAPHS, 256), jnp.float32),
        ],
        compiler_params=pltpu.CompilerParams(
            dimension_semantics=("arbitrary",)),
    )(x, h1lo, h1hi, z1lo, z1hi, batch3, w1lo, w1hi, b1, w2, b2,
      l0w, l1w, l2w, bsum)


def _prep_edges(edge_index, n_parts):
    src = edge_index[0]
    dst = edge_index[1]
    per = N_EDGES // n_parts
    nch = -(-per // K_EDGE)
    padn = nch * K_EDGE - per
    pad_src = jnp.zeros((n_parts, padn), jnp.int32)
    pad_dst = jnp.full((n_parts, padn), N_PAD, jnp.int32)
    src2 = jnp.concatenate([src.reshape(n_parts, per), pad_src], axis=1)
    dst2 = jnp.concatenate([dst.reshape(n_parts, per), pad_dst], axis=1)
    src3 = src2.reshape(n_parts, nch, K_EDGE)
    garb = HALF + (dst2 % GARB)
    dst3 = []
    for p in range(2):
        base = p * HALF
        rel = dst2 - base
        in_win = (rel >= 0) & (rel < HALF)
        dst3.append(jnp.where(in_win, rel, garb).reshape(n_parts, nch, K_EDGE))
    return src3, dst3[0], dst3[1]


def kernel(x, edge_index, batch,
           c0_W1, c0_b1, c0_g1, c0_be1, c0_W2, c0_b2, bn0_g, bn0_b,
           c1_W1, c1_b1, c1_g1, c1_be1, c1_W2, c1_b2, bn1_g, bn1_b,
           l0_W, l0_b, l1_W, l1_b, l2_W, l2_b):
    s = 1.0 / jnp.sqrt(1.0 + BN_EPS)

    a = s * c0_g1
    w1a = c0_W1 * a[None, :]
    b1a = (c0_b1 * a + c0_be1)[None, :]
    a = s * bn0_g
    w2a = c0_W2 * a[None, :]
    b2a = (c0_b2 * a + bn0_b)[None, :]

    a = s * c1_g1
    w1b = c1_W1 * a[None, :]
    b1b = (c1_b1 * a + c1_be1)[None, :]
    a = s * bn1_g
    w2b = c1_W2 * a[None, :]
    b2b = (c1_b2 * a + bn1_b)[None, :]

    bsum = (l0_b + l1_b + l2_b)[None, :]

    src32, dst32a, dst32b = _prep_edges(edge_index, 2 * NT)
    src16, dst16a, dst16b = _prep_edges(edge_index, NT)

    npad = N_PAD - N_NODES
    x_pad = jnp.concatenate([x, jnp.zeros((npad, x.shape[1]), jnp.float32)])
    zeros_tab = jnp.zeros((N_PAD, 128), jnp.float32)
    batch3 = jnp.concatenate(
        [batch, jnp.full((npad,), N_GRAPHS, jnp.int32)]).reshape(
            N_PAD // _R, 1, _R)

    aggr0 = _make_sc_aggr_edgesplit(N_PAD, 128, N_EDGES)
    z0_a, z0_b = aggr0(x_pad, zeros_tab, src32, dst32a, dst32b)

    h1_lo, h1_hi = _tc_mlp0(z0_a, z0_b, w1a, b1a, w2a, b2a)

    aggr1 = _make_sc_aggr_featsplit(N_PAD, 128, N_EDGES)
    z1_lo, z1_hi = aggr1(h1_lo, h1_hi, src16, dst16a, dst16b)

    return _tc_mlp1_pool(x_pad, h1_lo, h1_hi, z1_lo, z1_hi, batch3,
                         w1b[:128], w1b[128:], b1b, w2b, b2b,
                         l0_W, l1_W, l2_W, bsum)


# speedup vs baseline: 3.2723x; 1.2129x over previous
import functools

import jax
import jax.numpy as jnp
from jax import lax
from jax.experimental import pallas as pl
from jax.experimental.pallas import tpu as pltpu
from jax.experimental.pallas import tpu_sc as plsc

N_NODES = 10000
N_PAD = 10240
N_EDGES = 320000
N_GRAPHS = 64
BN_EPS = 1e-5

NT = 16
K_EDGE = 128
G_CH = 16
RPT = N_PAD // NT
ROW_CH = 64
N_RCH = RPT // ROW_CH


def _nch(per):
    n = -(-per // K_EDGE)
    return n + (-n) % G_CH


def _sc_phases(table_init, table_gather, out, src3, dst3, t,
               srcb, dstb, rows, acc, gsem, s, nch):
    row0 = s * RPT

    def stage(k, carry):
        r = row0 + k * ROW_CH
        pltpu.sync_copy(table_init.at[pl.ds(r, ROW_CH)],
                        rows.at[pl.ds(0, ROW_CH)])
        pltpu.sync_copy(rows.at[pl.ds(0, ROW_CH)], acc.at[pl.ds(r, ROW_CH)])
        return carry
    lax.fori_loop(0, N_RCH, stage, 0)
    plsc.subcore_barrier()

    def group(g, carry):
        pltpu.sync_copy(src3.at[t, pl.ds(g * G_CH, G_CH)], srcb)
        pltpu.sync_copy(dst3.at[t, pl.ds(g * G_CH, G_CH)], dstb)

        def chunk(l, c2):
            pltpu.async_copy(table_gather.at[srcb.at[l]], rows, gsem).wait()
            pltpu.sync_copy(rows, acc.at[dstb.at[l]], add=True)
            return c2
        lax.fori_loop(0, G_CH, chunk, 0)
        return carry
    lax.fori_loop(0, nch // G_CH, group, 0)
    plsc.subcore_barrier()

    def wout(k, carry):
        r = row0 + k * ROW_CH
        pltpu.sync_copy(acc.at[pl.ds(r, ROW_CH)], rows.at[pl.ds(0, ROW_CH)])
        pltpu.sync_copy(rows.at[pl.ds(0, ROW_CH)], out.at[pl.ds(r, ROW_CH)])
        return carry
    lax.fori_loop(0, N_RCH, wout, 0)


def _make_sc_aggr_edgesplit(n, d, e):
    per_tile = e // (2 * NT)
    nch = _nch(per_tile)

    mesh = plsc.VectorSubcoreMesh(core_axis_name="c", subcore_axis_name="s")

    @functools.partial(
        pl.kernel,
        out_type=(jax.ShapeDtypeStruct((n, d), jnp.float32),
                  jax.ShapeDtypeStruct((n, d), jnp.float32)),
        mesh=mesh,
        scratch_types=[
            pltpu.VMEM((G_CH, K_EDGE), jnp.int32),
            pltpu.VMEM((G_CH, K_EDGE), jnp.int32),
            pltpu.VMEM((K_EDGE, d), jnp.float32),
            pltpu.VMEM_SHARED((N_PAD, d), jnp.float32),
            pltpu.SemaphoreType.DMA,
        ],
    )
    def aggr(table, zeros, src3, dst3, out_a, out_b,
             srcb, dstb, rows, acc, gsem):
        c = lax.axis_index("c")
        s = lax.axis_index("s")
        wid = c * NT + s

        @pl.when(c == 0)
        def _():
            _sc_phases(table, table, out_a, src3, dst3, wid,
                       srcb, dstb, rows, acc, gsem, s, nch)

        @pl.when(c == 1)
        def _():
            _sc_phases(zeros, table, out_b, src3, dst3, wid,
                       srcb, dstb, rows, acc, gsem, s, nch)

    return aggr


def _make_sc_aggr_featsplit(n, d, e):
    per_tile = e // NT
    nch = _nch(per_tile)

    mesh = plsc.VectorSubcoreMesh(core_axis_name="c", subcore_axis_name="s")

    @functools.partial(
        pl.kernel,
        out_type=(jax.ShapeDtypeStruct((n, d), jnp.float32),
                  jax.ShapeDtypeStruct((n, d), jnp.float32)),
        mesh=mesh,
        scratch_types=[
            pltpu.VMEM((G_CH, K_EDGE), jnp.int32),
            pltpu.VMEM((G_CH, K_EDGE), jnp.int32),
            pltpu.VMEM((K_EDGE, d), jnp.float32),
            pltpu.VMEM_SHARED((N_PAD, d), jnp.float32),
            pltpu.SemaphoreType.DMA,
        ],
    )
    def aggr(tlo, thi, src3, dst3, out_lo, out_hi,
             srcb, dstb, rows, acc, gsem):
        c = lax.axis_index("c")
        s = lax.axis_index("s")

        @pl.when(c == 0)
        def _():
            _sc_phases(tlo, tlo, out_lo, src3, dst3, s,
                       srcb, dstb, rows, acc, gsem, s, nch)

        @pl.when(c == 1)
        def _():
            _sc_phases(thi, thi, out_hi, src3, dst3, s,
                       srcb, dstb, rows, acc, gsem, s, nch)

    return aggr


_R = 512


def _mlp0_body(za, zb, w1, b1, w2, b2, olo, ohi):
    z = za[...] + zb[...]
    t = jnp.dot(z, w1[...], preferred_element_type=jnp.float32) + b1[...]
    t = jnp.maximum(t, 0.0)
    h = jnp.dot(t, w2[...], preferred_element_type=jnp.float32) + b2[...]
    h = jnp.maximum(h, 0.0)
    olo[...] = h[:, :128]
    ohi[...] = h[:, 128:]


def _tc_mlp0(z_a, z_b, w1, b1, w2, b2):
    n = z_a.shape[0]
    grid = n // _R
    return pl.pallas_call(
        _mlp0_body,
        grid=(grid,),
        in_specs=[
            pl.BlockSpec((_R, 128), lambda i: (i, 0)),
            pl.BlockSpec((_R, 128), lambda i: (i, 0)),
            pl.BlockSpec((128, 256), lambda i: (0, 0)),
            pl.BlockSpec((1, 256), lambda i: (0, 0)),
            pl.BlockSpec((256, 256), lambda i: (0, 0)),
            pl.BlockSpec((1, 256), lambda i: (0, 0)),
        ],
        out_specs=[
            pl.BlockSpec((_R, 128), lambda i: (i, 0)),
            pl.BlockSpec((_R, 128), lambda i: (i, 0)),
        ],
        out_shape=[
            jax.ShapeDtypeStruct((n, 128), jnp.float32),
            jax.ShapeDtypeStruct((n, 128), jnp.float32),
        ],
    )(z_a, z_b, w1, b1, w2, b2)


def _mlp1_pool_body(x, h1lo, h1hi, z1lo, z1hi, bvec, w1lo, w1hi, b1, w2, b2,
                    l0w, l1w, l2w, bsum, out, p0, p1, p2):
    i = pl.program_id(0)
    nb = pl.num_programs(0)

    @pl.when(i == 0)
    def _():
        p0[...] = jnp.zeros_like(p0)
        p1[...] = jnp.zeros_like(p1)
        p2[...] = jnp.zeros_like(p2)

    t = (jnp.dot(z1lo[...], w1lo[...], preferred_element_type=jnp.float32)
         + jnp.dot(z1hi[...], w1hi[...], preferred_element_type=jnp.float32)
         + b1[...])
    t = jnp.maximum(t, 0.0)
    h2 = jnp.dot(t, w2[...], preferred_element_type=jnp.float32) + b2[...]
    h2 = jnp.maximum(h2, 0.0)

    b = bvec[0, 0, :]
    pm = (b[None, :] == lax.broadcasted_iota(jnp.int32, (N_GRAPHS, _R), 0)
          ).astype(jnp.float32)
    h1 = jnp.concatenate([h1lo[...], h1hi[...]], axis=1)
    p0[...] += jnp.dot(pm, x[...], preferred_element_type=jnp.float32)
    p1[...] += jnp.dot(pm, h1, preferred_element_type=jnp.float32)
    p2[...] += jnp.dot(pm, h2, preferred_element_type=jnp.float32)

    @pl.when(i == nb - 1)
    def _():
        out[...] = (
            jnp.dot(p0[...], l0w[...], preferred_element_type=jnp.float32)
            + jnp.dot(p1[...], l1w[...], preferred_element_type=jnp.float32)
            + jnp.dot(p2[...], l2w[...], preferred_element_type=jnp.float32)
            + bsum[...])


def _tc_mlp1_pool(x, h1lo, h1hi, z1lo, z1hi, batch3, w1lo, w1hi, b1, w2, b2,
                  l0w, l1w, l2w, bsum):
    n = x.shape[0]
    grid = n // _R
    return pl.pallas_call(
        _mlp1_pool_body,
        grid=(grid,),
        in_specs=[
            pl.BlockSpec((_R, 128), lambda i: (i, 0)),
            pl.BlockSpec((_R, 128), lambda i: (i, 0)),
            pl.BlockSpec((_R, 128), lambda i: (i, 0)),
            pl.BlockSpec((_R, 128), lambda i: (i, 0)),
            pl.BlockSpec((_R, 128), lambda i: (i, 0)),
            pl.BlockSpec((1, 1, _R), lambda i: (i, 0, 0)),
            pl.BlockSpec((128, 256), lambda i: (0, 0)),
            pl.BlockSpec((128, 256), lambda i: (0, 0)),
            pl.BlockSpec((1, 256), lambda i: (0, 0)),
            pl.BlockSpec((256, 256), lambda i: (0, 0)),
            pl.BlockSpec((1, 256), lambda i: (0, 0)),
            pl.BlockSpec((128, 128), lambda i: (0, 0)),
            pl.BlockSpec((256, 128), lambda i: (0, 0)),
            pl.BlockSpec((256, 128), lambda i: (0, 0)),
            pl.BlockSpec((1, 128), lambda i: (0, 0)),
        ],
        out_specs=pl.BlockSpec((N_GRAPHS, 128), lambda i: (0, 0)),
        out_shape=jax.ShapeDtypeStruct((N_GRAPHS, 128), jnp.float32),
        scratch_shapes=[
            pltpu.VMEM((N_GRAPHS, 128), jnp.float32),
            pltpu.VMEM((N_GRAPHS, 256), jnp.float32),
            pltpu.VMEM((N_GRAPHS, 256), jnp.float32),
        ],
        compiler_params=pltpu.CompilerParams(
            dimension_semantics=("arbitrary",)),
    )(x, h1lo, h1hi, z1lo, z1hi, batch3, w1lo, w1hi, b1, w2, b2,
      l0w, l1w, l2w, bsum)


def _prep_edges(edge_index, n_parts):
    src = edge_index[0]
    dst = edge_index[1]
    per = N_EDGES // n_parts
    nch = _nch(per)
    padn = nch * K_EDGE - per
    pad_src = jnp.zeros((n_parts, padn), jnp.int32)
    pad_dst = jnp.broadcast_to(
        N_NODES + (jnp.arange(padn, dtype=jnp.int32) % NT), (n_parts, padn))
    src2 = jnp.concatenate([src.reshape(n_parts, per), pad_src], axis=1)
    dst2 = jnp.concatenate([dst.reshape(n_parts, per), pad_dst], axis=1)
    return (src2.reshape(n_parts, nch, K_EDGE),
            dst2.reshape(n_parts, nch, K_EDGE))


def kernel(x, edge_index, batch,
           c0_W1, c0_b1, c0_g1, c0_be1, c0_W2, c0_b2, bn0_g, bn0_b,
           c1_W1, c1_b1, c1_g1, c1_be1, c1_W2, c1_b2, bn1_g, bn1_b,
           l0_W, l0_b, l1_W, l1_b, l2_W, l2_b):
    s = 1.0 / jnp.sqrt(1.0 + BN_EPS)

    a = s * c0_g1
    w1a = c0_W1 * a[None, :]
    b1a = (c0_b1 * a + c0_be1)[None, :]
    a = s * bn0_g
    w2a = c0_W2 * a[None, :]
    b2a = (c0_b2 * a + bn0_b)[None, :]

    a = s * c1_g1
    w1b = c1_W1 * a[None, :]
    b1b = (c1_b1 * a + c1_be1)[None, :]
    a = s * bn1_g
    w2b = c1_W2 * a[None, :]
    b2b = (c1_b2 * a + bn1_b)[None, :]

    bsum = (l0_b + l1_b + l2_b)[None, :]

    src32, dst32 = _prep_edges(edge_index, 2 * NT)
    src16, dst16 = _prep_edges(edge_index, NT)

    npad = N_PAD - N_NODES
    x_pad = jnp.concatenate([x, jnp.zeros((npad, x.shape[1]), jnp.float32)])
    zeros_tab = jnp.zeros((N_PAD, 128), jnp.float32)
    batch3 = jnp.concatenate(
        [batch, jnp.full((npad,), N_GRAPHS, jnp.int32)]).reshape(
            N_PAD // _R, 1, _R)

    aggr0 = _make_sc_aggr_edgesplit(N_PAD, 128, N_EDGES)
    z0_a, z0_b = aggr0(x_pad, zeros_tab, src32, dst32)

    h1_lo, h1_hi = _tc_mlp0(z0_a, z0_b, w1a, b1a, w2a, b2a)

    aggr1 = _make_sc_aggr_featsplit(N_PAD, 128, N_EDGES)
    z1_lo, z1_hi = aggr1(h1_lo, h1_hi, src16, dst16)

    return _tc_mlp1_pool(x_pad, h1_lo, h1_hi, z1_lo, z1_hi, batch3,
                         w1b[:128], w1b[128:], b1b, w2b, b2b,
                         l0_W, l1_W, l2_W, bsum)


# speedup vs baseline: 3.3157x; 1.0133x over previous
import functools

import jax
import jax.numpy as jnp
from jax import lax
from jax.experimental import pallas as pl
from jax.experimental.pallas import tpu as pltpu
from jax.experimental.pallas import tpu_sc as plsc

N_NODES = 10000
N_PAD = 10240
N_EDGES = 320000
N_GRAPHS = 64
BN_EPS = 1e-5

NT = 16
K_EDGE = 128
G_CH = 16
RPT = N_PAD // NT
ROW_CH = 128
N_RCH = RPT // ROW_CH


def _nch(per):
    n = -(-per // K_EDGE)
    return n + (-n) % G_CH


def _sc_phases(table_init, table_gather, out, src3, dst3, t,
               srcb, dstb, rows, acc, gsem, s, nch):
    row0 = s * RPT

    def stage(k, carry):
        r = row0 + k * ROW_CH
        pltpu.sync_copy(table_init.at[pl.ds(r, ROW_CH)],
                        acc.at[pl.ds(r, ROW_CH)])
        return carry
    lax.fori_loop(0, N_RCH, stage, 0)
    plsc.subcore_barrier()

    def group(g, carry):
        pltpu.sync_copy(src3.at[t, pl.ds(g * G_CH, G_CH)], srcb)
        pltpu.sync_copy(dst3.at[t, pl.ds(g * G_CH, G_CH)], dstb)

        def chunk(l, c2):
            pltpu.async_copy(table_gather.at[srcb.at[l]], rows, gsem).wait()
            pltpu.sync_copy(rows, acc.at[dstb.at[l]], add=True)
            return c2
        lax.fori_loop(0, G_CH, chunk, 0)
        return carry
    lax.fori_loop(0, nch // G_CH, group, 0)
    plsc.subcore_barrier()

    def wout(k, carry):
        r = row0 + k * ROW_CH
        pltpu.sync_copy(acc.at[pl.ds(r, ROW_CH)], out.at[pl.ds(r, ROW_CH)])
        return carry
    lax.fori_loop(0, N_RCH, wout, 0)


def _make_sc_aggr_edgesplit(n, d, e):
    per_tile = e // (2 * NT)
    nch = _nch(per_tile)

    mesh = plsc.VectorSubcoreMesh(core_axis_name="c", subcore_axis_name="s")

    @functools.partial(
        pl.kernel,
        out_type=(jax.ShapeDtypeStruct((n, d), jnp.float32),
                  jax.ShapeDtypeStruct((n, d), jnp.float32)),
        mesh=mesh,
        scratch_types=[
            pltpu.VMEM((G_CH, K_EDGE), jnp.int32),
            pltpu.VMEM((G_CH, K_EDGE), jnp.int32),
            pltpu.VMEM((K_EDGE, d), jnp.float32),
            pltpu.VMEM_SHARED((N_PAD, d), jnp.float32),
            pltpu.SemaphoreType.DMA,
        ],
    )
    def aggr(table, zeros, src3, dst3, out_a, out_b,
             srcb, dstb, rows, acc, gsem):
        c = lax.axis_index("c")
        s = lax.axis_index("s")
        wid = c * NT + s

        @pl.when(c == 0)
        def _():
            _sc_phases(table, table, out_a, src3, dst3, wid,
                       srcb, dstb, rows, acc, gsem, s, nch)

        @pl.when(c == 1)
        def _():
            _sc_phases(zeros, table, out_b, src3, dst3, wid,
                       srcb, dstb, rows, acc, gsem, s, nch)

    return aggr


def _make_sc_aggr_featsplit(n, d, e):
    per_tile = e // NT
    nch = _nch(per_tile)

    mesh = plsc.VectorSubcoreMesh(core_axis_name="c", subcore_axis_name="s")

    @functools.partial(
        pl.kernel,
        out_type=(jax.ShapeDtypeStruct((n, d), jnp.float32),
                  jax.ShapeDtypeStruct((n, d), jnp.float32)),
        mesh=mesh,
        scratch_types=[
            pltpu.VMEM((G_CH, K_EDGE), jnp.int32),
            pltpu.VMEM((G_CH, K_EDGE), jnp.int32),
            pltpu.VMEM((K_EDGE, d), jnp.float32),
            pltpu.VMEM_SHARED((N_PAD, d), jnp.float32),
            pltpu.SemaphoreType.DMA,
        ],
    )
    def aggr(tlo, thi, src3, dst3, out_lo, out_hi,
             srcb, dstb, rows, acc, gsem):
        c = lax.axis_index("c")
        s = lax.axis_index("s")

        @pl.when(c == 0)
        def _():
            _sc_phases(tlo, tlo, out_lo, src3, dst3, s,
                       srcb, dstb, rows, acc, gsem, s, nch)

        @pl.when(c == 1)
        def _():
            _sc_phases(thi, thi, out_hi, src3, dst3, s,
                       srcb, dstb, rows, acc, gsem, s, nch)

    return aggr


_R = 512


def _mlp0_body(za, zb, w1, b1, w2, b2, olo, ohi):
    z = za[...] + zb[...]
    t = jnp.dot(z, w1[...], preferred_element_type=jnp.float32) + b1[...]
    t = jnp.maximum(t, 0.0)
    h = jnp.dot(t, w2[...], preferred_element_type=jnp.float32) + b2[...]
    h = jnp.maximum(h, 0.0)
    olo[...] = h[:, :128]
    ohi[...] = h[:, 128:]


def _tc_mlp0(z_a, z_b, w1, b1, w2, b2):
    n = z_a.shape[0]
    grid = n // _R
    return pl.pallas_call(
        _mlp0_body,
        grid=(grid,),
        in_specs=[
            pl.BlockSpec((_R, 128), lambda i: (i, 0)),
            pl.BlockSpec((_R, 128), lambda i: (i, 0)),
            pl.BlockSpec((128, 256), lambda i: (0, 0)),
            pl.BlockSpec((1, 256), lambda i: (0, 0)),
            pl.BlockSpec((256, 256), lambda i: (0, 0)),
            pl.BlockSpec((1, 256), lambda i: (0, 0)),
        ],
        out_specs=[
            pl.BlockSpec((_R, 128), lambda i: (i, 0)),
            pl.BlockSpec((_R, 128), lambda i: (i, 0)),
        ],
        out_shape=[
            jax.ShapeDtypeStruct((n, 128), jnp.float32),
            jax.ShapeDtypeStruct((n, 128), jnp.float32),
        ],
    )(z_a, z_b, w1, b1, w2, b2)


def _mlp1_pool_body(x, h1lo, h1hi, z1lo, z1hi, bvec, w1lo, w1hi, b1, w2, b2,
                    l0w, l1w, l2w, bsum, out, p0, p1, p2):
    i = pl.program_id(0)
    nb = pl.num_programs(0)

    @pl.when(i == 0)
    def _():
        p0[...] = jnp.zeros_like(p0)
        p1[...] = jnp.zeros_like(p1)
        p2[...] = jnp.zeros_like(p2)

    t = (jnp.dot(z1lo[...], w1lo[...], preferred_element_type=jnp.float32)
         + jnp.dot(z1hi[...], w1hi[...], preferred_element_type=jnp.float32)
         + b1[...])
    t = jnp.maximum(t, 0.0)
    h2 = jnp.dot(t, w2[...], preferred_element_type=jnp.float32) + b2[...]
    h2 = jnp.maximum(h2, 0.0)

    b = bvec[0, 0, :]
    pm = (b[None, :] == lax.broadcasted_iota(jnp.int32, (N_GRAPHS, _R), 0)
          ).astype(jnp.float32)
    h1 = jnp.concatenate([h1lo[...], h1hi[...]], axis=1)
    p0[...] += jnp.dot(pm, x[...], preferred_element_type=jnp.float32)
    p1[...] += jnp.dot(pm, h1, preferred_element_type=jnp.float32)
    p2[...] += jnp.dot(pm, h2, preferred_element_type=jnp.float32)

    @pl.when(i == nb - 1)
    def _():
        out[...] = (
            jnp.dot(p0[...], l0w[...], preferred_element_type=jnp.float32)
            + jnp.dot(p1[...], l1w[...], preferred_element_type=jnp.float32)
            + jnp.dot(p2[...], l2w[...], preferred_element_type=jnp.float32)
            + bsum[...])


def _tc_mlp1_pool(x, h1lo, h1hi, z1lo, z1hi, batch3, w1lo, w1hi, b1, w2, b2,
                  l0w, l1w, l2w, bsum):
    n = x.shape[0]
    grid = n // _R
    return pl.pallas_call(
        _mlp1_pool_body,
        grid=(grid,),
        in_specs=[
            pl.BlockSpec((_R, 128), lambda i: (i, 0)),
            pl.BlockSpec((_R, 128), lambda i: (i, 0)),
            pl.BlockSpec((_R, 128), lambda i: (i, 0)),
            pl.BlockSpec((_R, 128), lambda i: (i, 0)),
            pl.BlockSpec((_R, 128), lambda i: (i, 0)),
            pl.BlockSpec((1, 1, _R), lambda i: (i, 0, 0)),
            pl.BlockSpec((128, 256), lambda i: (0, 0)),
            pl.BlockSpec((128, 256), lambda i: (0, 0)),
            pl.BlockSpec((1, 256), lambda i: (0, 0)),
            pl.BlockSpec((256, 256), lambda i: (0, 0)),
            pl.BlockSpec((1, 256), lambda i: (0, 0)),
            pl.BlockSpec((128, 128), lambda i: (0, 0)),
            pl.BlockSpec((256, 128), lambda i: (0, 0)),
            pl.BlockSpec((256, 128), lambda i: (0, 0)),
            pl.BlockSpec((1, 128), lambda i: (0, 0)),
        ],
        out_specs=pl.BlockSpec((N_GRAPHS, 128), lambda i: (0, 0)),
        out_shape=jax.ShapeDtypeStruct((N_GRAPHS, 128), jnp.float32),
        scratch_shapes=[
            pltpu.VMEM((N_GRAPHS, 128), jnp.float32),
            pltpu.VMEM((N_GRAPHS, 256), jnp.float32),
            pltpu.VMEM((N_GRAPHS, 256), jnp.float32),
        ],
        compiler_params=pltpu.CompilerParams(
            dimension_semantics=("arbitrary",)),
    )(x, h1lo, h1hi, z1lo, z1hi, batch3, w1lo, w1hi, b1, w2, b2,
      l0w, l1w, l2w, bsum)


def _prep_edges(edge_index, n_parts):
    src = edge_index[0]
    dst = edge_index[1]
    per = N_EDGES // n_parts
    nch = _nch(per)
    padn = nch * K_EDGE - per
    pad_src = jnp.zeros((n_parts, padn), jnp.int32)
    pad_dst = jnp.broadcast_to(
        N_NODES + (jnp.arange(padn, dtype=jnp.int32) % NT), (n_parts, padn))
    src2 = jnp.concatenate([src.reshape(n_parts, per), pad_src], axis=1)
    dst2 = jnp.concatenate([dst.reshape(n_parts, per), pad_dst], axis=1)
    return (src2.reshape(n_parts, nch, K_EDGE),
            dst2.reshape(n_parts, nch, K_EDGE))


def kernel(x, edge_index, batch,
           c0_W1, c0_b1, c0_g1, c0_be1, c0_W2, c0_b2, bn0_g, bn0_b,
           c1_W1, c1_b1, c1_g1, c1_be1, c1_W2, c1_b2, bn1_g, bn1_b,
           l0_W, l0_b, l1_W, l1_b, l2_W, l2_b):
    s = 1.0 / jnp.sqrt(1.0 + BN_EPS)

    a = s * c0_g1
    w1a = c0_W1 * a[None, :]
    b1a = (c0_b1 * a + c0_be1)[None, :]
    a = s * bn0_g
    w2a = c0_W2 * a[None, :]
    b2a = (c0_b2 * a + bn0_b)[None, :]

    a = s * c1_g1
    w1b = c1_W1 * a[None, :]
    b1b = (c1_b1 * a + c1_be1)[None, :]
    a = s * bn1_g
    w2b = c1_W2 * a[None, :]
    b2b = (c1_b2 * a + bn1_b)[None, :]

    bsum = (l0_b + l1_b + l2_b)[None, :]

    src32, dst32 = _prep_edges(edge_index, 2 * NT)
    src16, dst16 = _prep_edges(edge_index, NT)

    npad = N_PAD - N_NODES
    x_pad = jnp.concatenate([x, jnp.zeros((npad, x.shape[1]), jnp.float32)])
    zeros_tab = jnp.zeros((N_PAD, 128), jnp.float32)
    batch3 = jnp.concatenate(
        [batch, jnp.full((npad,), N_GRAPHS, jnp.int32)]).reshape(
            N_PAD // _R, 1, _R)

    aggr0 = _make_sc_aggr_edgesplit(N_PAD, 128, N_EDGES)
    z0_a, z0_b = aggr0(x_pad, zeros_tab, src32, dst32)

    h1_lo, h1_hi = _tc_mlp0(z0_a, z0_b, w1a, b1a, w2a, b2a)

    aggr1 = _make_sc_aggr_featsplit(N_PAD, 128, N_EDGES)
    z1_lo, z1_hi = aggr1(h1_lo, h1_hi, src16, dst16)

    return _tc_mlp1_pool(x_pad, h1_lo, h1_hi, z1_lo, z1_hi, batch3,
                         w1b[:128], w1b[128:], b1b, w2b, b2b,
                         l0_W, l1_W, l2_W, bsum)


# speedup vs baseline: 8.0024x; 2.4134x over previous
import functools

import jax
import jax.numpy as jnp
from jax import lax
from jax.experimental import pallas as pl
from jax.experimental.pallas import tpu as pltpu
from jax.experimental.pallas import tpu_sc as plsc

N_NODES = 10000
N_PAD = 10240
N_EDGES = 320000
N_GRAPHS = 64
BN_EPS = 1e-5

NT = 16
K_EDGE = 128
G_CH = 16
RPT = N_PAD // NT
ROW_CH = 128
N_RCH = RPT // ROW_CH


def _nch(per):
    n = -(-per // K_EDGE)
    return n + (-n) % G_CH


def _sc_phases(table_init, table_gather, out, src3, dst3, t,
               srcb, dstb, rows, acc, gsems, s, nch):
    row0 = s * RPT

    def stage(k, carry):
        r = row0 + k * ROW_CH
        pltpu.sync_copy(table_init.at[pl.ds(r, ROW_CH)],
                        acc.at[pl.ds(r, ROW_CH)])
        return carry
    lax.fori_loop(0, N_RCH, stage, 0)
    plsc.subcore_barrier()

    def fire_g(l, b):
        pltpu.async_copy(table_gather.at[srcb.at[l]], rows[b], gsems[b])

    def wait_g(b):
        pltpu.make_async_copy(table_gather.at[srcb.at[0]], rows[b],
                              gsems[b]).wait()

    def scat(l, b):
        pltpu.sync_copy(rows[b], acc.at[dstb.at[l]], add=True)

    def group(g, carry):
        pltpu.sync_copy(src3.at[t, pl.ds(g * G_CH, G_CH)], srcb)
        pltpu.sync_copy(dst3.at[t, pl.ds(g * G_CH, G_CH)], dstb)
        fire_g(0, 0)

        def duo(i, c2):
            l0 = 2 * i
            wait_g(0)
            fire_g(l0 + 1, 1)
            scat(l0, 0)
            wait_g(1)
            fire_g(l0 + 2, 0)
            scat(l0 + 1, 1)
            return c2
        lax.fori_loop(0, G_CH // 2 - 1, duo, 0)
        wait_g(0)
        fire_g(G_CH - 1, 1)
        scat(G_CH - 2, 0)
        wait_g(1)
        scat(G_CH - 1, 1)
        return carry
    lax.fori_loop(0, nch // G_CH, group, 0)
    plsc.subcore_barrier()

    def wout(k, carry):
        r = row0 + k * ROW_CH
        pltpu.sync_copy(acc.at[pl.ds(r, ROW_CH)], out.at[pl.ds(r, ROW_CH)])
        return carry
    lax.fori_loop(0, N_RCH, wout, 0)


def _make_sc_aggr_edgesplit(n, d, e):
    per_tile = e // (2 * NT)
    nch = _nch(per_tile)

    mesh = plsc.VectorSubcoreMesh(core_axis_name="c", subcore_axis_name="s")

    @functools.partial(
        pl.kernel,
        out_type=(jax.ShapeDtypeStruct((n, d), jnp.float32),
                  jax.ShapeDtypeStruct((n, d), jnp.float32)),
        mesh=mesh,
        scratch_types=[
            pltpu.VMEM((G_CH, K_EDGE), jnp.int32),
            pltpu.VMEM((G_CH, K_EDGE), jnp.int32),
            [pltpu.VMEM((K_EDGE, d), jnp.float32)] * 2,
            pltpu.VMEM_SHARED((N_PAD, d), jnp.float32),
            [pltpu.SemaphoreType.DMA] * 2,
        ],
    )
    def aggr(table, zeros, src3, dst3, out_a, out_b,
             srcb, dstb, rows, acc, gsems):
        c = lax.axis_index("c")
        s = lax.axis_index("s")
        wid = c * NT + s

        @pl.when(c == 0)
        def _():
            _sc_phases(table, table, out_a, src3, dst3, wid,
                       srcb, dstb, rows, acc, gsems, s, nch)

        @pl.when(c == 1)
        def _():
            _sc_phases(zeros, table, out_b, src3, dst3, wid,
                       srcb, dstb, rows, acc, gsems, s, nch)

    return aggr


def _make_sc_aggr_featsplit(n, d, e):
    per_tile = e // NT
    nch = _nch(per_tile)

    mesh = plsc.VectorSubcoreMesh(core_axis_name="c", subcore_axis_name="s")

    @functools.partial(
        pl.kernel,
        out_type=(jax.ShapeDtypeStruct((n, d), jnp.float32),
                  jax.ShapeDtypeStruct((n, d), jnp.float32)),
        mesh=mesh,
        scratch_types=[
            pltpu.VMEM((G_CH, K_EDGE), jnp.int32),
            pltpu.VMEM((G_CH, K_EDGE), jnp.int32),
            [pltpu.VMEM((K_EDGE, d), jnp.float32)] * 2,
            pltpu.VMEM_SHARED((N_PAD, d), jnp.float32),
            [pltpu.SemaphoreType.DMA] * 2,
        ],
    )
    def aggr(tlo, thi, src3, dst3, out_lo, out_hi,
             srcb, dstb, rows, acc, gsems):
        c = lax.axis_index("c")
        s = lax.axis_index("s")

        @pl.when(c == 0)
        def _():
            _sc_phases(tlo, tlo, out_lo, src3, dst3, s,
                       srcb, dstb, rows, acc, gsems, s, nch)

        @pl.when(c == 1)
        def _():
            _sc_phases(thi, thi, out_hi, src3, dst3, s,
                       srcb, dstb, rows, acc, gsems, s, nch)

    return aggr


_R = 512


def _mlp0_body(za, zb, w1, b1, w2, b2, olo, ohi):
    z = za[...] + zb[...]
    t = jnp.dot(z, w1[...], preferred_element_type=jnp.float32) + b1[...]
    t = jnp.maximum(t, 0.0)
    h = jnp.dot(t, w2[...], preferred_element_type=jnp.float32) + b2[...]
    h = jnp.maximum(h, 0.0)
    olo[...] = h[:, :128]
    ohi[...] = h[:, 128:]


def _tc_mlp0(z_a, z_b, w1, b1, w2, b2):
    n = z_a.shape[0]
    grid = n // _R
    return pl.pallas_call(
        _mlp0_body,
        grid=(grid,),
        in_specs=[
            pl.BlockSpec((_R, 128), lambda i: (i, 0)),
            pl.BlockSpec((_R, 128), lambda i: (i, 0)),
            pl.BlockSpec((128, 256), lambda i: (0, 0)),
            pl.BlockSpec((1, 256), lambda i: (0, 0)),
            pl.BlockSpec((256, 256), lambda i: (0, 0)),
            pl.BlockSpec((1, 256), lambda i: (0, 0)),
        ],
        out_specs=[
            pl.BlockSpec((_R, 128), lambda i: (i, 0)),
            pl.BlockSpec((_R, 128), lambda i: (i, 0)),
        ],
        out_shape=[
            jax.ShapeDtypeStruct((n, 128), jnp.float32),
            jax.ShapeDtypeStruct((n, 128), jnp.float32),
        ],
    )(z_a, z_b, w1, b1, w2, b2)


def _mlp1_pool_body(x, h1lo, h1hi, z1lo, z1hi, bvec, w1lo, w1hi, b1, w2, b2,
                    l0w, l1w, l2w, bsum, out, p0, p1, p2):
    i = pl.program_id(0)
    nb = pl.num_programs(0)

    @pl.when(i == 0)
    def _():
        p0[...] = jnp.zeros_like(p0)
        p1[...] = jnp.zeros_like(p1)
        p2[...] = jnp.zeros_like(p2)

    t = (jnp.dot(z1lo[...], w1lo[...], preferred_element_type=jnp.float32)
         + jnp.dot(z1hi[...], w1hi[...], preferred_element_type=jnp.float32)
         + b1[...])
    t = jnp.maximum(t, 0.0)
    h2 = jnp.dot(t, w2[...], preferred_element_type=jnp.float32) + b2[...]
    h2 = jnp.maximum(h2, 0.0)

    b = bvec[0, 0, :]
    pm = (b[None, :] == lax.broadcasted_iota(jnp.int32, (N_GRAPHS, _R), 0)
          ).astype(jnp.float32)
    h1 = jnp.concatenate([h1lo[...], h1hi[...]], axis=1)
    p0[...] += jnp.dot(pm, x[...], preferred_element_type=jnp.float32)
    p1[...] += jnp.dot(pm, h1, preferred_element_type=jnp.float32)
    p2[...] += jnp.dot(pm, h2, preferred_element_type=jnp.float32)

    @pl.when(i == nb - 1)
    def _():
        out[...] = (
            jnp.dot(p0[...], l0w[...], preferred_element_type=jnp.float32)
            + jnp.dot(p1[...], l1w[...], preferred_element_type=jnp.float32)
            + jnp.dot(p2[...], l2w[...], preferred_element_type=jnp.float32)
            + bsum[...])


def _tc_mlp1_pool(x, h1lo, h1hi, z1lo, z1hi, batch3, w1lo, w1hi, b1, w2, b2,
                  l0w, l1w, l2w, bsum):
    n = x.shape[0]
    grid = n // _R
    return pl.pallas_call(
        _mlp1_pool_body,
        grid=(grid,),
        in_specs=[
            pl.BlockSpec((_R, 128), lambda i: (i, 0)),
            pl.BlockSpec((_R, 128), lambda i: (i, 0)),
            pl.BlockSpec((_R, 128), lambda i: (i, 0)),
            pl.BlockSpec((_R, 128), lambda i: (i, 0)),
            pl.BlockSpec((_R, 128), lambda i: (i, 0)),
            pl.BlockSpec((1, 1, _R), lambda i: (i, 0, 0)),
            pl.BlockSpec((128, 256), lambda i: (0, 0)),
            pl.BlockSpec((128, 256), lambda i: (0, 0)),
            pl.BlockSpec((1, 256), lambda i: (0, 0)),
            pl.BlockSpec((256, 256), lambda i: (0, 0)),
            pl.BlockSpec((1, 256), lambda i: (0, 0)),
            pl.BlockSpec((128, 128), lambda i: (0, 0)),
            pl.BlockSpec((256, 128), lambda i: (0, 0)),
            pl.BlockSpec((256, 128), lambda i: (0, 0)),
            pl.BlockSpec((1, 128), lambda i: (0, 0)),
        ],
        out_specs=pl.BlockSpec((N_GRAPHS, 128), lambda i: (0, 0)),
        out_shape=jax.ShapeDtypeStruct((N_GRAPHS, 128), jnp.float32),
        scratch_shapes=[
            pltpu.VMEM((N_GRAPHS, 128), jnp.float32),
            pltpu.VMEM((N_GRAPHS, 256), jnp.float32),
            pltpu.VMEM((N_GRAPHS, 256), jnp.float32),
        ],
        compiler_params=pltpu.CompilerParams(
            dimension_semantics=("arbitrary",)),
    )(x, h1lo, h1hi, z1lo, z1hi, batch3, w1lo, w1hi, b1, w2, b2,
      l0w, l1w, l2w, bsum)


def _prep_edges(edge_index, n_parts):
    src = edge_index[0]
    dst = edge_index[1]
    per = N_EDGES // n_parts
    nch = _nch(per)
    padn = nch * K_EDGE - per
    pad_src = jnp.broadcast_to(
        (jnp.arange(padn, dtype=jnp.int32) * 37) % N_NODES, (n_parts, padn))
    pad_dst = jnp.broadcast_to(
        N_NODES + (jnp.arange(padn, dtype=jnp.int32) % NT), (n_parts, padn))
    src2 = jnp.concatenate([src.reshape(n_parts, per), pad_src], axis=1)
    dst2 = jnp.concatenate([dst.reshape(n_parts, per), pad_dst], axis=1)
    return (src2.reshape(n_parts, nch, K_EDGE),
            dst2.reshape(n_parts, nch, K_EDGE))


def kernel(x, edge_index, batch,
           c0_W1, c0_b1, c0_g1, c0_be1, c0_W2, c0_b2, bn0_g, bn0_b,
           c1_W1, c1_b1, c1_g1, c1_be1, c1_W2, c1_b2, bn1_g, bn1_b,
           l0_W, l0_b, l1_W, l1_b, l2_W, l2_b):
    s = 1.0 / jnp.sqrt(1.0 + BN_EPS)

    a = s * c0_g1
    w1a = c0_W1 * a[None, :]
    b1a = (c0_b1 * a + c0_be1)[None, :]
    a = s * bn0_g
    w2a = c0_W2 * a[None, :]
    b2a = (c0_b2 * a + bn0_b)[None, :]

    a = s * c1_g1
    w1b = c1_W1 * a[None, :]
    b1b = (c1_b1 * a + c1_be1)[None, :]
    a = s * bn1_g
    w2b = c1_W2 * a[None, :]
    b2b = (c1_b2 * a + bn1_b)[None, :]

    bsum = (l0_b + l1_b + l2_b)[None, :]

    src32, dst32 = _prep_edges(edge_index, 2 * NT)
    src16, dst16 = _prep_edges(edge_index, NT)

    npad = N_PAD - N_NODES
    x_pad = jnp.concatenate([x, jnp.zeros((npad, x.shape[1]), jnp.float32)])
    zeros_tab = jnp.zeros((N_PAD, 128), jnp.float32)
    batch3 = jnp.concatenate(
        [batch, jnp.full((npad,), N_GRAPHS, jnp.int32)]).reshape(
            N_PAD // _R, 1, _R)

    aggr0 = _make_sc_aggr_edgesplit(N_PAD, 128, N_EDGES)
    z0_a, z0_b = aggr0(x_pad, zeros_tab, src32, dst32)

    h1_lo, h1_hi = _tc_mlp0(z0_a, z0_b, w1a, b1a, w2a, b2a)

    aggr1 = _make_sc_aggr_featsplit(N_PAD, 128, N_EDGES)
    z1_lo, z1_hi = aggr1(h1_lo, h1_hi, src16, dst16)

    return _tc_mlp1_pool(x_pad, h1_lo, h1_hi, z1_lo, z1_hi, batch3,
                         w1b[:128], w1b[128:], b1b, w2b, b2b,
                         l0_W, l1_W, l2_W, bsum)
